# baseline (speedup 1.0000x reference)
import functools

import jax
import jax.numpy as jnp
from jax import lax
from jax.experimental import pallas as pl
from jax.experimental.pallas import tpu as pltpu
from jax.experimental.pallas import tpu_sc as plsc

_NG = 1024
_D = 128
_N = 100000
_T = 0.1
_CH = 128
_NFULL = _N // _CH
_TAIL = _N - _NFULL * _CH
_NCHUNK = _NFULL + 1
_IDROWS = 800
_JUNK = _NG
_AROWS = 1152
_RPT = _AROWS // 16
_NW = 32
_MAXC = _IDROWS // _NW
_HBLK = 100


def _sc_segsum(x1, x2, ids1, ids2, zacc,
               s1_out, s2_out,
               idb1, idb2, xbuf0, xbuf1, sem0, sem1,
               ssem0, ssem1, ssem2, ssem3,
               acc1, acc2):
    cid = lax.axis_index("c")
    sid = lax.axis_index("s")
    wid = sid * 2 + cid

    r0 = sid * _RPT
    pltpu.sync_copy(zacc.at[pl.ds(r0, _RPT)], acc1.at[pl.ds(r0, _RPT)])
    pltpu.sync_copy(zacc.at[pl.ds(r0, _RPT)], acc2.at[pl.ds(r0, _RPT)])
    plsc.subcore_barrier()

    start = _MAXC * wid
    count = jnp.clip(_NCHUNK - start, 0, _MAXC)

    pltpu.sync_copy(ids1.at[wid], idb1)
    pltpu.sync_copy(ids2.at[wid], idb2)

    bufs = (xbuf0, xbuf1)
    sems = (sem0, sem1)

    def process(x, idb, acc, scsems):
        scat = {}

        def load(j):
            c = start + j
            row0 = jnp.where(c == _NFULL, _N - _CH, c * _CH)
            pltpu.async_copy(x.at[pl.ds(row0, _CH)], bufs[j % 2], sems[j % 2])

        def wait_buf(j):
            pltpu.make_async_copy(x.at[pl.ds(0, _CH)], bufs[j % 2],
                                  sems[j % 2]).wait()

        @pl.when(0 < count)
        def _():
            load(0)

        for j in range(_MAXC):
            @pl.when(j < count)
            def _():
                @pl.when(j + 1 < count)
                def _():
                    if j >= 1:
                        scat[j - 1].wait()
                    load(j + 1)

                wait_buf(j)
                scat[j] = pltpu.async_copy(bufs[j % 2], acc.at[idb.at[j]],
                                           scsems[j % 2], add=True)

                @pl.when(j + 1 == count)
                def _():
                    if j >= 1:
                        scat[j - 1].wait()
                    scat[j].wait()

    process(x1, idb1, acc1, (ssem0, ssem1))
    process(x2, idb2, acc2, (ssem2, ssem3))

    plsc.subcore_barrier()

    pltpu.sync_copy(acc1.at[pl.ds(r0, _RPT)], s1_out.at[cid, pl.ds(r0, _RPT)])
    pltpu.sync_copy(acc2.at[pl.ds(r0, _RPT)], s2_out.at[cid, pl.ds(r0, _RPT)])


_sc_call = functools.partial(
    pl.kernel,
    _sc_segsum,
    out_type=[
        jax.ShapeDtypeStruct((2, _AROWS, _D), jnp.float32),
        jax.ShapeDtypeStruct((2, _AROWS, _D), jnp.float32),
    ],
    mesh=plsc.VectorSubcoreMesh(core_axis_name="c", subcore_axis_name="s"),
    scratch_types=[
        pltpu.VMEM((_MAXC, _CH), jnp.int32),
        pltpu.VMEM((_MAXC, _CH), jnp.int32),
        pltpu.VMEM((_CH, _D), jnp.float32),
        pltpu.VMEM((_CH, _D), jnp.float32),
        pltpu.SemaphoreType.DMA,
        pltpu.SemaphoreType.DMA,
        pltpu.SemaphoreType.DMA,
        pltpu.SemaphoreType.DMA,
        pltpu.SemaphoreType.DMA,
        pltpu.SemaphoreType.DMA,
        pltpu.VMEM_SHARED((_AROWS, _D), jnp.float32),
        pltpu.VMEM_SHARED((_AROWS, _D), jnp.float32),
    ],
)()


def _histogram(h):
    f32 = jnp.float32
    bf16 = jnp.bfloat16
    hcol = lax.broadcasted_iota(jnp.int32, (_NG // _D, 1), 0)
    lcol = lax.broadcasted_iota(jnp.int32, (_D, 1), 0)

    def body(bidx, cnt2d):
        row = h[pl.ds(bidx, 1), :]
        at = (lax.shift_right_logical(row, 7) == hcol).astype(bf16)
        bt = ((row & 127) == lcol).astype(bf16)
        return cnt2d + lax.dot_general(
            at, bt, (((1,), (1,)), ((), ())),
            preferred_element_type=f32)

    cnt2d = lax.fori_loop(0, _HBLK, body, jnp.zeros((_NG // _D, _D), f32))
    giota = lax.broadcasted_iota(jnp.int32, (_NG, 1), 0)
    hrow = lax.broadcasted_iota(jnp.int32, (1, _NG // _D), 1)
    lrow = lax.broadcasted_iota(jnp.int32, (1, _D), 1)
    p = (lax.shift_right_logical(giota, 7) == hrow).astype(bf16)
    r = ((giota & 127) == lrow).astype(f32)
    tmp = lax.dot_general(p, cnt2d.astype(bf16), (((1,), (0,)), ((), ())),
                          preferred_element_type=f32)
    return jnp.sum(tmp * r, axis=1, keepdims=True)


def _counts_body(h1, h2, out):
    out[:, 0:1] = _histogram(h1)
    out[:, 1:2] = _histogram(h2)


def _tc_body(s1, s2, cnts, w1t, b1, w2t, b2, out):
    f32 = jnp.float32
    hi = jax.lax.Precision.HIGHEST

    def graph_emb(s, cnt):
        acc = s[0, :_NG, :] + s[1, :_NG, :]
        xg = acc / jnp.maximum(cnt, 1.0)
        hh = lax.dot_general(xg, w1t[...], (((1,), (0,)), ((), ())),
                             precision=hi, preferred_element_type=f32)
        hh = hh + b1[0:1, :]
        hh = hh * (1.0 / (1.0 + jnp.exp(-hh)))
        g = lax.dot_general(hh, w2t[...], (((1,), (0,)), ((), ())),
                            precision=hi, preferred_element_type=f32)
        return g + b2[0:1, :]

    g1 = graph_emb(s1[...], cnts[:, 0:1])
    g2 = graph_emb(s2[...], cnts[:, 1:2])
    sim = lax.dot_general(g1, g2, (((1,), (1,)), ((), ())),
                          precision=hi, preferred_element_type=f32) * (1.0 / _T)
    m = jnp.max(sim, axis=1, keepdims=True)
    lse = jnp.log(jnp.sum(jnp.exp(sim - m), axis=1, keepdims=True)) + m
    rows = lax.broadcasted_iota(jnp.int32, (_NG, _NG), 0)
    cols = lax.broadcasted_iota(jnp.int32, (_NG, _NG), 1)
    diag = jnp.sum(jnp.where(rows == cols, sim, 0.0), axis=1, keepdims=True)
    out[...] = jnp.sum(lse - diag, axis=(0, 1), keepdims=True) * (1.0 / _NG)


def kernel(x1, x2, node2graph1, node2graph2, W1, b1, W2, b2):
    pad = jnp.full((_IDROWS * _CH - _N,), _JUNK, dtype=jnp.int32)
    junk96 = jnp.full((_CH - _TAIL,), _JUNK, dtype=jnp.int32)

    def arrange(ids):
        ids = ids.astype(jnp.int32)
        return jnp.concatenate(
            [ids[: _NFULL * _CH], junk96, ids[_NFULL * _CH:],
             pad[: _IDROWS * _CH - _NCHUNK * _CH]])

    flat1 = arrange(node2graph1)
    flat2 = arrange(node2graph2)
    ids1 = flat1.reshape(_NW, _MAXC, _CH)
    ids2 = flat2.reshape(_NW, _MAXC, _CH)
    zacc = jnp.zeros((_AROWS, _D), jnp.float32)

    s1, s2 = _sc_call(x1, x2, ids1, ids2, zacc)

    h1 = flat1.reshape(_HBLK, _NG)
    h2 = flat2.reshape(_HBLK, _NG)

    cnts = pl.pallas_call(
        _counts_body,
        out_shape=jax.ShapeDtypeStruct((_NG, 2), jnp.float32),
    )(h1, h2)

    out = pl.pallas_call(
        _tc_body,
        out_shape=jax.ShapeDtypeStruct((1, 1), jnp.float32),
    )(s1, s2, cnts, W1.T, b1.reshape(1, _D), W2.T, b2.reshape(1, _D))
    return out[0, 0]


# speedup vs baseline: 1.0925x; 1.0925x over previous
import functools

import jax
import jax.numpy as jnp
from jax import lax
from jax.experimental import pallas as pl
from jax.experimental.pallas import tpu as pltpu
from jax.experimental.pallas import tpu_sc as plsc

_NG = 1024
_D = 128
_N = 100000
_T = 0.1
_CH = 128
_NFULL = _N // _CH
_TAIL = _N - _NFULL * _CH
_NCHUNK = _NFULL + 1
_IDROWS = 800
_JUNK = _NG
_AROWS = 1152
_RPT = _AROWS // 16
_NW = 32
_MAXC = _IDROWS // _NW
_HBLK = 100


def _sc_segsum(x1, x2, ids1, ids2, zacc,
               s1_out, s2_out,
               idb1, idb2, xbuf0, xbuf1, xbuf2, xbuf3,
               sem0, sem1, sem2, sem3,
               acc1, acc2):
    cid = lax.axis_index("c")
    sid = lax.axis_index("s")
    wid = sid * 2 + cid

    r0 = sid * _RPT
    pltpu.sync_copy(zacc.at[pl.ds(r0, _RPT)], acc1.at[pl.ds(r0, _RPT)])
    pltpu.sync_copy(zacc.at[pl.ds(r0, _RPT)], acc2.at[pl.ds(r0, _RPT)])
    plsc.subcore_barrier()

    start = _MAXC * wid
    count = jnp.clip(_NCHUNK - start, 0, _MAXC)

    pltpu.sync_copy(ids1.at[wid], idb1)
    pltpu.sync_copy(ids2.at[wid], idb2)

    chains = ((x1, idb1, acc1, (xbuf0, xbuf1), (sem0, sem1)),
              (x2, idb2, acc2, (xbuf2, xbuf3), (sem2, sem3)))

    def load(chain, j):
        x, _, _, cbufs, csems = chain
        c = start + j
        row0 = jnp.where(c == _NFULL, _N - _CH, c * _CH)
        pltpu.async_copy(x.at[pl.ds(row0, _CH)], cbufs[j % 2], csems[j % 2])

    def wait_buf(chain, j):
        x, _, _, cbufs, csems = chain
        pltpu.make_async_copy(x.at[pl.ds(0, _CH)], cbufs[j % 2],
                              csems[j % 2]).wait()

    @pl.when(0 < count)
    def _():
        for chain in chains:
            load(chain, 0)

    for j in range(_MAXC):
        @pl.when(j < count)
        def _():
            for chain in chains:
                @pl.when(j + 1 < count)
                def _():
                    load(chain, j + 1)

                _, idb, acc, cbufs, _ = chain
                wait_buf(chain, j)
                pltpu.sync_copy(cbufs[j % 2], acc.at[idb.at[j]], add=True)

    plsc.subcore_barrier()

    pltpu.sync_copy(acc1.at[pl.ds(r0, _RPT)], s1_out.at[cid, pl.ds(r0, _RPT)])
    pltpu.sync_copy(acc2.at[pl.ds(r0, _RPT)], s2_out.at[cid, pl.ds(r0, _RPT)])


_sc_call = functools.partial(
    pl.kernel,
    _sc_segsum,
    out_type=[
        jax.ShapeDtypeStruct((2, _AROWS, _D), jnp.float32),
        jax.ShapeDtypeStruct((2, _AROWS, _D), jnp.float32),
    ],
    mesh=plsc.VectorSubcoreMesh(core_axis_name="c", subcore_axis_name="s"),
    scratch_types=[
        pltpu.VMEM((_MAXC, _CH), jnp.int32),
        pltpu.VMEM((_MAXC, _CH), jnp.int32),
        pltpu.VMEM((_CH, _D), jnp.float32),
        pltpu.VMEM((_CH, _D), jnp.float32),
        pltpu.VMEM((_CH, _D), jnp.float32),
        pltpu.VMEM((_CH, _D), jnp.float32),
        pltpu.SemaphoreType.DMA,
        pltpu.SemaphoreType.DMA,
        pltpu.SemaphoreType.DMA,
        pltpu.SemaphoreType.DMA,
        pltpu.VMEM_SHARED((_AROWS, _D), jnp.float32),
        pltpu.VMEM_SHARED((_AROWS, _D), jnp.float32),
    ],
)()


def _histogram(h):
    f32 = jnp.float32
    bf16 = jnp.bfloat16
    hcol = lax.broadcasted_iota(jnp.int32, (_NG // _D, 1), 0)
    lcol = lax.broadcasted_iota(jnp.int32, (_D, 1), 0)

    def body(bidx, cnt2d):
        row = h[pl.ds(bidx, 1), :]
        at = (lax.shift_right_logical(row, 7) == hcol).astype(bf16)
        bt = ((row & 127) == lcol).astype(bf16)
        return cnt2d + lax.dot_general(
            at, bt, (((1,), (1,)), ((), ())),
            preferred_element_type=f32)

    cnt2d = lax.fori_loop(0, _HBLK, body, jnp.zeros((_NG // _D, _D), f32))
    giota = lax.broadcasted_iota(jnp.int32, (_NG, 1), 0)
    hrow = lax.broadcasted_iota(jnp.int32, (1, _NG // _D), 1)
    lrow = lax.broadcasted_iota(jnp.int32, (1, _D), 1)
    p = (lax.shift_right_logical(giota, 7) == hrow).astype(bf16)
    r = ((giota & 127) == lrow).astype(f32)
    tmp = lax.dot_general(p, cnt2d.astype(bf16), (((1,), (0,)), ((), ())),
                          preferred_element_type=f32)
    return jnp.sum(tmp * r, axis=1, keepdims=True)


def _counts_body(h1, h2, out):
    out[:, 0:1] = _histogram(h1)
    out[:, 1:2] = _histogram(h2)


def _tc_body(s1, s2, cnts, w1t, b1, w2t, b2, out):
    f32 = jnp.float32
    hi = jax.lax.Precision.HIGHEST

    def graph_emb(s, cnt):
        acc = s[0, :_NG, :] + s[1, :_NG, :]
        xg = acc / jnp.maximum(cnt, 1.0)
        hh = lax.dot_general(xg, w1t[...], (((1,), (0,)), ((), ())),
                             precision=hi, preferred_element_type=f32)
        hh = hh + b1[0:1, :]
        hh = hh * (1.0 / (1.0 + jnp.exp(-hh)))
        g = lax.dot_general(hh, w2t[...], (((1,), (0,)), ((), ())),
                            precision=hi, preferred_element_type=f32)
        return g + b2[0:1, :]

    g1 = graph_emb(s1[...], cnts[:, 0:1])
    g2 = graph_emb(s2[...], cnts[:, 1:2])
    sim = lax.dot_general(g1, g2, (((1,), (1,)), ((), ())),
                          precision=hi, preferred_element_type=f32) * (1.0 / _T)
    m = jnp.max(sim, axis=1, keepdims=True)
    lse = jnp.log(jnp.sum(jnp.exp(sim - m), axis=1, keepdims=True)) + m
    rows = lax.broadcasted_iota(jnp.int32, (_NG, _NG), 0)
    cols = lax.broadcasted_iota(jnp.int32, (_NG, _NG), 1)
    diag = jnp.sum(jnp.where(rows == cols, sim, 0.0), axis=1, keepdims=True)
    out[...] = jnp.sum(lse - diag, axis=(0, 1), keepdims=True) * (1.0 / _NG)


def kernel(x1, x2, node2graph1, node2graph2, W1, b1, W2, b2):
    pad = jnp.full((_IDROWS * _CH - _N,), _JUNK, dtype=jnp.int32)
    junk96 = jnp.full((_CH - _TAIL,), _JUNK, dtype=jnp.int32)

    def arrange(ids):
        ids = ids.astype(jnp.int32)
        return jnp.concatenate(
            [ids[: _NFULL * _CH], junk96, ids[_NFULL * _CH:],
             pad[: _IDROWS * _CH - _NCHUNK * _CH]])

    flat1 = arrange(node2graph1)
    flat2 = arrange(node2graph2)
    ids1 = flat1.reshape(_NW, _MAXC, _CH)
    ids2 = flat2.reshape(_NW, _MAXC, _CH)
    zacc = jnp.zeros((_AROWS, _D), jnp.float32)

    s1, s2 = _sc_call(x1, x2, ids1, ids2, zacc)

    h1 = flat1.reshape(_HBLK, _NG)
    h2 = flat2.reshape(_HBLK, _NG)

    cnts = pl.pallas_call(
        _counts_body,
        out_shape=jax.ShapeDtypeStruct((_NG, 2), jnp.float32),
    )(h1, h2)

    out = pl.pallas_call(
        _tc_body,
        out_shape=jax.ShapeDtypeStruct((1, 1), jnp.float32),
    )(s1, s2, cnts, W1.T, b1.reshape(1, _D), W2.T, b2.reshape(1, _D))
    return out[0, 0]


# speedup vs baseline: 1.0962x; 1.0035x over previous
import functools

import jax
import jax.numpy as jnp
from jax import lax
from jax.experimental import pallas as pl
from jax.experimental.pallas import tpu as pltpu
from jax.experimental.pallas import tpu_sc as plsc

_NG = 1024
_D = 128
_N = 100000
_T = 0.1
_CH = 128
_NFULL = _N // _CH
_TAIL = _N - _NFULL * _CH
_NCHUNK = _NFULL + 1
_IDROWS = 800
_JUNK = _NG
_AROWS = 1152
_RPT = _AROWS // 16
_NW = 32
_MAXC = _IDROWS // _NW
_HBLK = 100


def _sc_segsum(x1, x2, ids1, ids2, zacc,
               s1_out, s2_out,
               idb1, idb2, xbuf0, xbuf1, xbuf2, xbuf3,
               sem0, sem1, sem2, sem3,
               acc1, acc2):
    cid = lax.axis_index("c")
    sid = lax.axis_index("s")
    wid = sid * 2 + cid

    r0 = sid * _RPT
    pltpu.sync_copy(zacc.at[pl.ds(r0, _RPT)], acc1.at[pl.ds(r0, _RPT)])
    pltpu.sync_copy(zacc.at[pl.ds(r0, _RPT)], acc2.at[pl.ds(r0, _RPT)])
    plsc.subcore_barrier()

    start = _MAXC * wid
    count = jnp.clip(_NCHUNK - start, 0, _MAXC)

    pltpu.sync_copy(ids1.at[wid], idb1)
    pltpu.sync_copy(ids2.at[wid], idb2)

    chains = ((x1, idb1, acc1, (xbuf0, xbuf1), (sem0, sem1)),
              (x2, idb2, acc2, (xbuf2, xbuf3), (sem2, sem3)))

    def load(chain, j):
        x, _, _, cbufs, csems = chain
        c = start + j
        row0 = jnp.where(c == _NFULL, _N - _CH, c * _CH)
        pltpu.async_copy(x.at[pl.ds(row0, _CH)], cbufs[j % 2], csems[j % 2])

    def wait_buf(chain, j):
        x, _, _, cbufs, csems = chain
        pltpu.make_async_copy(x.at[pl.ds(0, _CH)], cbufs[j % 2],
                              csems[j % 2]).wait()

    @pl.when(0 < count)
    def _():
        for chain in chains:
            load(chain, 0)

    for j in range(_MAXC):
        @pl.when(j < count)
        def _():
            for chain in chains:
                @pl.when(j + 1 < count)
                def _():
                    load(chain, j + 1)

                _, idb, acc, cbufs, _ = chain
                wait_buf(chain, j)
                pltpu.sync_copy(cbufs[j % 2], acc.at[idb.at[j]], add=True)

    plsc.subcore_barrier()

    pltpu.sync_copy(acc1.at[pl.ds(r0, _RPT)], s1_out.at[cid, pl.ds(r0, _RPT)])
    pltpu.sync_copy(acc2.at[pl.ds(r0, _RPT)], s2_out.at[cid, pl.ds(r0, _RPT)])


_sc_call = functools.partial(
    pl.kernel,
    _sc_segsum,
    out_type=[
        jax.ShapeDtypeStruct((2, _AROWS, _D), jnp.float32),
        jax.ShapeDtypeStruct((2, _AROWS, _D), jnp.float32),
    ],
    mesh=plsc.VectorSubcoreMesh(core_axis_name="c", subcore_axis_name="s"),
    scratch_types=[
        pltpu.VMEM((_MAXC, _CH), jnp.int32),
        pltpu.VMEM((_MAXC, _CH), jnp.int32),
        pltpu.VMEM((_CH, _D), jnp.float32),
        pltpu.VMEM((_CH, _D), jnp.float32),
        pltpu.VMEM((_CH, _D), jnp.float32),
        pltpu.VMEM((_CH, _D), jnp.float32),
        pltpu.SemaphoreType.DMA,
        pltpu.SemaphoreType.DMA,
        pltpu.SemaphoreType.DMA,
        pltpu.SemaphoreType.DMA,
        pltpu.VMEM_SHARED((_AROWS, _D), jnp.float32),
        pltpu.VMEM_SHARED((_AROWS, _D), jnp.float32),
    ],
)()


def _histogram(h):
    f32 = jnp.float32
    bf16 = jnp.bfloat16
    hcol = lax.broadcasted_iota(jnp.int32, (_NG // _D, 1), 0)
    lcol = lax.broadcasted_iota(jnp.int32, (_D, 1), 0)

    def body(bidx, cnt2d):
        row = h[pl.ds(bidx, 1), :]
        at = (lax.shift_right_logical(row, 7) == hcol).astype(bf16)
        bt = ((row & 127) == lcol).astype(bf16)
        return cnt2d + lax.dot_general(
            at, bt, (((1,), (1,)), ((), ())),
            preferred_element_type=f32)

    cnt2d = lax.fori_loop(0, _HBLK, body, jnp.zeros((_NG // _D, _D), f32))
    giota = lax.broadcasted_iota(jnp.int32, (_NG, 1), 0)
    hrow = lax.broadcasted_iota(jnp.int32, (1, _NG // _D), 1)
    lrow = lax.broadcasted_iota(jnp.int32, (1, _D), 1)
    p = (lax.shift_right_logical(giota, 7) == hrow).astype(f32)
    r = ((giota & 127) == lrow).astype(f32)
    tmp = lax.dot_general(p, cnt2d, (((1,), (0,)), ((), ())),
                          precision=jax.lax.Precision.HIGHEST,
                          preferred_element_type=f32)
    return jnp.sum(tmp * r, axis=1, keepdims=True)


def _counts_body(h1, h2, out):
    out[:, 0:1] = _histogram(h1)
    out[:, 1:2] = _histogram(h2)


def _tc_body(s1, s2, cnts, w1t, b1, w2t, b2, out):
    f32 = jnp.float32
    hi = jax.lax.Precision.HIGHEST

    def graph_emb(s, cnt):
        acc = s[0, :_NG, :] + s[1, :_NG, :]
        xg = acc / jnp.maximum(cnt, 1.0)
        hh = lax.dot_general(xg, w1t[...], (((1,), (0,)), ((), ())),
                             precision=hi, preferred_element_type=f32)
        hh = hh + b1[0:1, :]
        hh = hh * (1.0 / (1.0 + jnp.exp(-hh)))
        g = lax.dot_general(hh, w2t[...], (((1,), (0,)), ((), ())),
                            precision=hi, preferred_element_type=f32)
        return g + b2[0:1, :]

    g1 = graph_emb(s1[...], cnts[:, 0:1])
    g2 = graph_emb(s2[...], cnts[:, 1:2])
    sim = lax.dot_general(g1, g2, (((1,), (1,)), ((), ())),
                          precision=hi, preferred_element_type=f32) * (1.0 / _T)
    m = jnp.max(sim, axis=1, keepdims=True)
    lse = jnp.log(jnp.sum(jnp.exp(sim - m), axis=1, keepdims=True)) + m
    rows = lax.broadcasted_iota(jnp.int32, (_NG, _NG), 0)
    cols = lax.broadcasted_iota(jnp.int32, (_NG, _NG), 1)
    diag = jnp.sum(jnp.where(rows == cols, sim, 0.0), axis=1, keepdims=True)
    out[...] = jnp.sum(lse - diag, axis=(0, 1), keepdims=True) * (1.0 / _NG)


def kernel(x1, x2, node2graph1, node2graph2, W1, b1, W2, b2):
    pad = jnp.full((_IDROWS * _CH - _N,), _JUNK, dtype=jnp.int32)
    junk96 = jnp.full((_CH - _TAIL,), _JUNK, dtype=jnp.int32)

    def arrange(ids):
        ids = ids.astype(jnp.int32)
        return jnp.concatenate(
            [ids[: _NFULL * _CH], junk96, ids[_NFULL * _CH:],
             pad[: _IDROWS * _CH - _NCHUNK * _CH]])

    flat1 = arrange(node2graph1)
    flat2 = arrange(node2graph2)
    ids1 = flat1.reshape(_NW, _MAXC, _CH)
    ids2 = flat2.reshape(_NW, _MAXC, _CH)
    zacc = jnp.zeros((_AROWS, _D), jnp.float32)

    s1, s2 = _sc_call(x1, x2, ids1, ids2, zacc)

    h1 = flat1.reshape(_HBLK, _NG)
    h2 = flat2.reshape(_HBLK, _NG)

    cnts = pl.pallas_call(
        _counts_body,
        out_shape=jax.ShapeDtypeStruct((_NG, 2), jnp.float32),
    )(h1, h2)

    out = pl.pallas_call(
        _tc_body,
        out_shape=jax.ShapeDtypeStruct((1, 1), jnp.float32),
    )(s1, s2, cnts, W1.T, b1.reshape(1, _D), W2.T, b2.reshape(1, _D))
    return out[0, 0]
